# Initial kernel scaffold; baseline (speedup 1.0000x reference)
import jax
import jax.numpy as jnp
from jax.experimental import pallas as pl


def kernel(xn, xe_src, xe_dst, W):
    raise NotImplementedError("write your pallas kernel here")


# speedup vs baseline: 2.1271x; 2.1271x over previous
import functools

import jax
import jax.numpy as jnp
from jax import lax
from jax.experimental import pallas as pl
from jax.experimental.pallas import tpu as pltpu
from jax.experimental.pallas import tpu_sc as plsc

N, E, D = 10000, 320000, 128
NC, NS, L = 2, 16, 16
NW = NC * NS
EPW = E // NW
K = 400
NCHUNK = EPW // K

_mesh = plsc.VectorSubcoreMesh(core_axis_name="c", subcore_axis_name="s")


def _bcast_lane(vec, l):
  return lax.gather(
      vec,
      jnp.full((L, 1), l, dtype=jnp.int32),
      lax.GatherDimensionNumbers(
          offset_dims=(), collapsed_slice_dims=(0,), start_index_map=(0,)),
      slice_sizes=(1,),
      mode=lax.GatherScatterMode.PROMISE_IN_BOUNDS)


@functools.partial(
    pl.kernel,
    mesh=_mesh,
    out_type=[
        jax.ShapeDtypeStruct((E, D), jnp.float32),
        jax.ShapeDtypeStruct((E, D), jnp.float32),
    ],
    scratch_types=[
        pltpu.VMEM((K,), jnp.int32),
        pltpu.VMEM((K,), jnp.int32),
        pltpu.VMEM((K,), jnp.float32),
        pltpu.VMEM((K, D), jnp.float32),
        pltpu.VMEM((K, D), jnp.float32),
        pltpu.SemaphoreType.DMA,
    ],
)
def _n2e(xn_hbm, src_hbm, dst_hbm, w_hbm, grad_hbm, ave_hbm,
         src_idx, dst_idx, w_v, s_rows, d_rows, sem):
  wid = lax.axis_index("s") * NC + lax.axis_index("c")
  wbase = wid * EPW

  def chunk_body(i, carry):
    base = wbase + i * K
    pltpu.sync_copy(src_hbm.at[pl.ds(base, K)], src_idx)
    pltpu.sync_copy(dst_hbm.at[pl.ds(base, K)], dst_idx)
    pltpu.sync_copy(w_hbm.at[pl.ds(base, K)], w_v)
    cp_s = pltpu.async_copy(xn_hbm.at[src_idx], s_rows, sem)
    cp_d = pltpu.async_copy(xn_hbm.at[dst_idx], d_rows, sem)
    cp_s.wait()
    cp_d.wait()

    def group_body(g, c2):
      wg = w_v[pl.ds(g * L, L)]

      def lane_body(l, c3):
        e = g * L + l
        wv = _bcast_lane(wg, l)
        wh = wv * 0.5
        for j in range(D // L):
          sl = pl.ds(j * L, L)
          s = s_rows[e, sl]
          d = d_rows[e, sl]
          s_rows[e, sl] = wv * (s - d)
          d_rows[e, sl] = wh * (s + d)
        return c3

      lax.fori_loop(0, L, lane_body, 0, unroll=False)
      return c2

    lax.fori_loop(0, K // L, group_body, 0, unroll=False)
    pltpu.sync_copy(s_rows, grad_hbm.at[pl.ds(base, K)])
    pltpu.sync_copy(d_rows, ave_hbm.at[pl.ds(base, K)])
    return carry

  lax.fori_loop(0, NCHUNK, chunk_body, 0, unroll=False)


def kernel(xn, xe_src, xe_dst, W):
  src = xe_src.astype(jnp.int32)
  dst = xe_dst.astype(jnp.int32)
  w = W.reshape(-1).astype(jnp.float32)
  grad, ave = _n2e(xn, src, dst, w)
  return grad, ave


# speedup vs baseline: 6.1325x; 2.8831x over previous
import functools

import jax
import jax.numpy as jnp
from jax import lax
from jax.experimental import pallas as pl
from jax.experimental.pallas import tpu as pltpu
from jax.experimental.pallas import tpu_sc as plsc

N, E, D = 10000, 320000, 128
NC, NS, L = 2, 16, 16
NW = NC * NS
EPW = E // NW
K = 80
NCHUNK = EPW // K
NGROUP = K // L

_mesh = plsc.VectorSubcoreMesh(core_axis_name="c", subcore_axis_name="s")

_BCAST_DNUMS = lax.GatherDimensionNumbers(
    offset_dims=(), collapsed_slice_dims=(0,), start_index_map=(0,))


def _bcast_lane(vec, l):
  return lax.gather(
      vec,
      jnp.full((L, 1), l, dtype=jnp.int32),
      _BCAST_DNUMS,
      slice_sizes=(1,),
      mode=lax.GatherScatterMode.PROMISE_IN_BOUNDS)


@functools.partial(
    pl.kernel,
    mesh=_mesh,
    out_type=[
        jax.ShapeDtypeStruct((E, D), jnp.float32),
        jax.ShapeDtypeStruct((E, D), jnp.float32),
    ],
    scratch_types=[
        pltpu.VMEM((2, K), jnp.int32),
        pltpu.VMEM((2, K), jnp.int32),
        pltpu.VMEM((2, K), jnp.float32),
        pltpu.VMEM((2, K, D), jnp.float32),
        pltpu.VMEM((2, K, D), jnp.float32),
        pltpu.SemaphoreType.DMA,
        pltpu.SemaphoreType.DMA,
        pltpu.SemaphoreType.DMA,
        pltpu.SemaphoreType.DMA,
        pltpu.SemaphoreType.DMA,
        pltpu.SemaphoreType.DMA,
    ],
)
def _n2e(xn_hbm, src_hbm, dst_hbm, w_hbm, grad_hbm, ave_hbm,
         si, di, wv, sr, dr, g0, g1, i0, i1, o0, o1):
  gsem = (g0, g1)
  isem = (i0, i1)
  osem = (o0, o1)
  wid = lax.axis_index("s") * NC + lax.axis_index("c")
  wbase = wid * EPW

  def drain_out(b):
    pltpu.make_async_copy(sr.at[b], grad_hbm.at[pl.ds(0, K)], osem[b]).wait()
    pltpu.make_async_copy(dr.at[b], ave_hbm.at[pl.ds(0, K)], osem[b]).wait()

  def drain_idx(b):
    pltpu.make_async_copy(src_hbm.at[pl.ds(0, K)], si.at[b], isem[b]).wait()
    pltpu.make_async_copy(dst_hbm.at[pl.ds(0, K)], di.at[b], isem[b]).wait()
    pltpu.make_async_copy(w_hbm.at[pl.ds(0, K)], wv.at[b], isem[b]).wait()

  def drain_gather(b):
    pltpu.make_async_copy(xn_hbm.at[si.at[b]], sr.at[b], gsem[b]).wait()
    pltpu.make_async_copy(xn_hbm.at[di.at[b]], dr.at[b], gsem[b]).wait()

  def issue_idx(c, b):
    base = wbase + c * K
    pltpu.async_copy(src_hbm.at[pl.ds(base, K)], si.at[b], isem[b])
    pltpu.async_copy(dst_hbm.at[pl.ds(base, K)], di.at[b], isem[b])

  def issue_w(c, b):
    base = wbase + c * K
    pltpu.async_copy(w_hbm.at[pl.ds(base, K)], wv.at[b], isem[b])

  def issue_gather(b):
    pltpu.async_copy(xn_hbm.at[si.at[b]], sr.at[b], gsem[b])
    pltpu.async_copy(xn_hbm.at[di.at[b]], dr.at[b], gsem[b])

  def issue_out(c, b):
    base = wbase + c * K
    pltpu.async_copy(sr.at[b], grad_hbm.at[pl.ds(base, K)], osem[b])
    pltpu.async_copy(dr.at[b], ave_hbm.at[pl.ds(base, K)], osem[b])

  def compute(b):
    def group_body(g, c2):
      wg = wv[b, pl.ds(g * L, L)]
      for l in range(L):
        e = g * L + l
        wfull = _bcast_lane(wg, l)
        whalf = wfull * 0.5
        for j in range(D // L):
          sl = pl.ds(j * L, L)
          s = sr[b, e, sl]
          d = dr[b, e, sl]
          sr[b, e, sl] = wfull * (s - d)
          dr[b, e, sl] = whalf * (s + d)
      return c2

    lax.fori_loop(0, NGROUP, group_body, 0, unroll=False)

  def section(c, b):
    ob = 1 - b

    @pl.when(c < NCHUNK)
    def _():
      @pl.when(c > 0)
      def _():
        drain_out(ob)

      @pl.when(c + 1 < NCHUNK)
      def _():
        drain_idx(ob)
        issue_gather(ob)

      drain_gather(b)

      @pl.when(c + 2 < NCHUNK)
      def _():
        issue_idx(c + 2, b)

      compute(b)

      issue_out(c, b)

      @pl.when(c + 2 < NCHUNK)
      def _():
        issue_w(c + 2, b)

  base0 = wbase
  pltpu.sync_copy(src_hbm.at[pl.ds(base0, K)], si.at[0])
  pltpu.sync_copy(dst_hbm.at[pl.ds(base0, K)], di.at[0])
  pltpu.sync_copy(w_hbm.at[pl.ds(base0, K)], wv.at[0])
  issue_gather(0)
  issue_idx(1, 1)
  issue_w(1, 1)

  def outer_body(io, carry):
    section(2 * io, 0)
    section(2 * io + 1, 1)
    return carry

  lax.fori_loop(0, (NCHUNK + 1) // 2, outer_body, 0, unroll=False)

  drain_out((NCHUNK - 1) % 2)


def kernel(xn, xe_src, xe_dst, W):
  src = xe_src.astype(jnp.int32)
  dst = xe_dst.astype(jnp.int32)
  w = W.reshape(-1).astype(jnp.float32)
  grad, ave = _n2e(xn, src, dst, w)
  return grad, ave


# speedup vs baseline: 6.3326x; 1.0326x over previous
import functools

import jax
import jax.numpy as jnp
from jax import lax
from jax.experimental import pallas as pl
from jax.experimental.pallas import tpu as pltpu
from jax.experimental.pallas import tpu_sc as plsc

N, E, D = 10000, 320000, 128
NC, NS, L = 2, 16, 16
NW = NC * NS
K = 160
NCHUNK_ALL = E // K
NCHUNK_LO = NCHUNK_ALL // NW
NREM = NCHUNK_ALL - NCHUNK_LO * NW
NGROUP = K // L

_mesh = plsc.VectorSubcoreMesh(core_axis_name="c", subcore_axis_name="s")

_BCAST_DNUMS = lax.GatherDimensionNumbers(
    offset_dims=(), collapsed_slice_dims=(0,), start_index_map=(0,))


def _bcast_lane(vec, l):
  return lax.gather(
      vec,
      jnp.full((L, 1), l, dtype=jnp.int32),
      _BCAST_DNUMS,
      slice_sizes=(1,),
      mode=lax.GatherScatterMode.PROMISE_IN_BOUNDS)


@functools.partial(
    pl.kernel,
    mesh=_mesh,
    out_type=[
        jax.ShapeDtypeStruct((E, D), jnp.float32),
        jax.ShapeDtypeStruct((E, D), jnp.float32),
    ],
    scratch_types=[
        pltpu.VMEM((K,), jnp.int32),
        pltpu.VMEM((K,), jnp.int32),
        pltpu.VMEM((K,), jnp.int32),
        pltpu.VMEM((K,), jnp.int32),
        pltpu.VMEM((K,), jnp.float32),
        pltpu.VMEM((K,), jnp.float32),
        pltpu.VMEM((K, D), jnp.float32),
        pltpu.VMEM((K, D), jnp.float32),
        pltpu.VMEM((K, D), jnp.float32),
        pltpu.VMEM((K, D), jnp.float32),
        pltpu.SemaphoreType.DMA,
        pltpu.SemaphoreType.DMA,
        pltpu.SemaphoreType.DMA,
        pltpu.SemaphoreType.DMA,
        pltpu.SemaphoreType.DMA,
        pltpu.SemaphoreType.DMA,
    ],
)
def _n2e(xn_hbm, src_hbm, dst_hbm, w_hbm, grad_hbm, ave_hbm,
         si0, si1, di0, di1, wv0, wv1, sr0, sr1, dr0, dr1,
         g0, g1, i0, i1, o0, o1):
  si = (si0, si1)
  di = (di0, di1)
  wv = (wv0, wv1)
  sr = (sr0, sr1)
  dr = (dr0, dr1)
  gsem = (g0, g1)
  isem = (i0, i1)
  osem = (o0, o1)
  wid = lax.axis_index("s") * NC + lax.axis_index("c")
  nchunk = NCHUNK_LO + jnp.where(wid < NREM, 1, 0)

  def cbase(c):
    return (wid + NW * c) * K

  def drain_out(b):
    pltpu.make_async_copy(sr[b], grad_hbm.at[pl.ds(0, K)], osem[b]).wait()
    pltpu.make_async_copy(dr[b], ave_hbm.at[pl.ds(0, K)], osem[b]).wait()

  def drain_idx(b):
    pltpu.make_async_copy(src_hbm.at[pl.ds(0, K)], si[b], isem[b]).wait()
    pltpu.make_async_copy(dst_hbm.at[pl.ds(0, K)], di[b], isem[b]).wait()
    pltpu.make_async_copy(w_hbm.at[pl.ds(0, K)], wv[b], isem[b]).wait()

  def drain_gather(b):
    pltpu.make_async_copy(xn_hbm.at[si[b]], sr[b], gsem[b]).wait()
    pltpu.make_async_copy(xn_hbm.at[di[b]], dr[b], gsem[b]).wait()

  def issue_idx(c, b):
    base = cbase(c)
    pltpu.async_copy(src_hbm.at[pl.ds(base, K)], si[b], isem[b])
    pltpu.async_copy(dst_hbm.at[pl.ds(base, K)], di[b], isem[b])

  def issue_w(c, b):
    base = cbase(c)
    pltpu.async_copy(w_hbm.at[pl.ds(base, K)], wv[b], isem[b])

  def issue_gather(b):
    pltpu.async_copy(xn_hbm.at[si[b]], sr[b], gsem[b])
    pltpu.async_copy(xn_hbm.at[di[b]], dr[b], gsem[b])

  def issue_out(c, b):
    base = cbase(c)
    pltpu.async_copy(sr[b], grad_hbm.at[pl.ds(base, K)], osem[b])
    pltpu.async_copy(dr[b], ave_hbm.at[pl.ds(base, K)], osem[b])

  def compute(b):
    def group_body(g, c2):
      wg = wv[b][pl.ds(g * L, L)]
      for l in range(L):
        e = g * L + l
        wfull = _bcast_lane(wg, l)
        whalf = wfull * 0.5
        for j in range(D // L):
          sl = pl.ds(j * L, L)
          s = sr[b][e, sl]
          d = dr[b][e, sl]
          sr[b][e, sl] = wfull * (s - d)
          dr[b][e, sl] = whalf * (s + d)
      return c2

    lax.fori_loop(0, NGROUP, group_body, 0, unroll=False)

  def section(c, b):
    ob = 1 - b

    @pl.when(c < nchunk)
    def _():
      @pl.when(c > 0)
      def _():
        drain_out(ob)

      @pl.when(c + 1 < nchunk)
      def _():
        drain_idx(ob)
        issue_gather(ob)

      drain_gather(b)

      @pl.when(c + 2 < nchunk)
      def _():
        issue_idx(c + 2, b)

      compute(b)

      issue_out(c, b)

      @pl.when(c + 2 < nchunk)
      def _():
        issue_w(c + 2, b)

  base0 = cbase(0)
  pltpu.sync_copy(src_hbm.at[pl.ds(base0, K)], si[0])
  pltpu.sync_copy(dst_hbm.at[pl.ds(base0, K)], di[0])
  pltpu.sync_copy(w_hbm.at[pl.ds(base0, K)], wv[0])
  issue_gather(0)
  issue_idx(1, 1)
  issue_w(1, 1)

  def outer_body(io, carry):
    section(2 * io, 0)
    section(2 * io + 1, 1)
    return carry

  lax.fori_loop(0, (NCHUNK_LO + 1 + 1) // 2, outer_body, 0, unroll=False)

  @pl.when(nchunk % 2 == 1)
  def _():
    drain_out(0)

  @pl.when(nchunk % 2 == 0)
  def _():
    drain_out(1)


def kernel(xn, xe_src, xe_dst, W):
  src = xe_src.astype(jnp.int32)
  dst = xe_dst.astype(jnp.int32)
  w = W.reshape(-1).astype(jnp.float32)
  grad, ave = _n2e(xn, src, dst, w)
  return grad, ave
